# Initial kernel scaffold; baseline (speedup 1.0000x reference)
import jax
import jax.numpy as jnp
from jax.experimental import pallas as pl


def kernel(boxes, scores):
    raise NotImplementedError("write your pallas kernel here")


# speedup vs baseline: 3.7432x; 3.7432x over previous
import jax
import jax.numpy as jnp
from jax import lax
from jax.experimental import pallas as pl
from jax.experimental.pallas import tpu as pltpu
from jax.experimental.pallas import tpu_sc as plsc

N = 20000
NPAD = 20480
TOPK = 400
CAND = 512
DET = 200
SCORE_THRESH = 0.01
NMS_THRESH = 0.5
NSUB = 16
PER_SUB = NPAD // NSUB
VPS = PER_SUB // 16

_f32 = jnp.float32
_i32 = jnp.int32
NEG_INF = float("-inf")
THRESH_BITS = 0x3C23D70A


def _excl(mask):
  m = mask.astype(_i32)
  return plsc.cumsum(m) - m, m


def _sc_body(sbits_hbm, boxes_hbm, osc_hbm, obx_hbm, obt_hbm,
             bits_v, hist_v, tgt_v, histrb_v,
             selb_v, seli_v, candb_v, candi_v, rowbuf_v, rank_v, obuf_v,
             ibuf_v, sbuf_v, gath_v, colb_v,
             hist_sh, counts_sh, candb_sh, candi_sh, sorted_sh, sem):
  sid = lax.axis_index("s")
  base = sid * PER_SUB
  ones16 = jnp.ones((16,), _i32)
  zeros16 = jnp.zeros((16,), _i32)
  iota = lax.iota(_i32, 16)
  tb = _i32(THRESH_BITS)

  pltpu.sync_copy(sbits_hbm.at[pl.ds(base, PER_SUB)], bits_v)
  for g in range(16):
    hist_v[g] = zeros16
  pltpu.sync_copy(hist_v.at[pl.ds(0, 4)], hist_sh.at[pl.ds(4 * sid, 4)])
  plsc.subcore_barrier()

  prefix = _i32(0)
  rem = _i32(0)
  kk = _i32(0)

  for r in range(4):
    shift = 24 - 8 * r
    for g in range(16):
      hist_v[g] = zeros16

    def p1(i, _):
      b = bits_v[pl.ds(i * 16, 16)]
      m = b > tb
      if r > 0:
        m = m & ((b >> (shift + 8)) == (prefix >> (shift + 8)))
      bucket = (b >> shift) & 255
      plsc.addupdate_scatter(hist_v, [bucket >> 4, bucket & 15], ones16,
                             mask=m)
      return 0
    lax.fori_loop(0, VPS, p1, 0)

    tgt_v[...] = iota + 16 * r
    pltpu.sync_copy(hist_v, hist_sh.at[tgt_v], add=True)
    plsc.subcore_barrier()
    pltpu.sync_copy(hist_sh.at[pl.ds(16 * r, 16)], histrb_v)

    if r == 0:
      total = _i32(0)
      for g in range(16):
        total = total + jnp.sum(histrb_v[g])
      kk = jnp.minimum(_i32(TOPK), total)
      rem = kk

    cum = _i32(0)
    bsel = _i32(0)
    sub = _i32(0)
    for g in range(15, -1, -1):
      row = histrb_v[g]
      rrow = lax.rev(row, (0,))
      rcum = plsc.cumsum(rrow)
      excl = rcum - rrow
      hit = ((cum + excl) < rem) & ((cum + rcum) >= rem)
      bsel = bsel + jnp.sum(jnp.where(hit, 16 * g + 15 - iota, 0))
      sub = sub + jnp.sum(jnp.where(hit, cum + excl, 0))
      cum = cum + jnp.sum(row)

    prefix = prefix | (bsel << shift)
    rem = rem - sub

  t_bits = prefix
  rem_eq = rem

  def p2(i, carry):
    g, e = carry
    b = bits_v[pl.ds(i * 16, 16)]
    valid = b > tb
    g = g + jnp.sum((valid & (b > t_bits)).astype(_i32))
    e = e + jnp.sum((valid & (b == t_bits)).astype(_i32))
    return g, e
  gt_w, eq_w = lax.fori_loop(0, VPS, p2, (_i32(0), _i32(0)))

  cvec = jnp.where(iota == 0, gt_w, jnp.where(iota == 1, eq_w, 0))
  tgt_v[...] = cvec
  pltpu.sync_copy(tgt_v, counts_sh.at[sid])
  plsc.subcore_barrier()
  pltpu.sync_copy(counts_sh, histrb_v)

  gt_all = plsc.load_gather(histrb_v, [iota, zeros16])
  eq_all = plsc.load_gather(histrb_v, [iota, ones16])
  eq_excl = plsc.cumsum(eq_all) - eq_all
  take_all = jnp.clip(rem_eq - eq_excl, 0, eq_all)
  c_all = gt_all + take_all
  c_excl = plsc.cumsum(c_all) - c_all
  my_off = jnp.sum(jnp.where(iota == sid, c_excl, 0))
  my_take = jnp.sum(jnp.where(iota == sid, take_all, 0))

  def p4(i, carry):
    nsel, taken = carry
    b = bits_v[pl.ds(i * 16, 16)]
    valid = b > tb
    gt = valid & (b > t_bits)
    eq = valid & (b == t_bits)
    eq_x, eq_i = _excl(eq)
    sel = gt | (eq & ((taken + eq_x) < my_take))
    pos_x, sel_i = _excl(sel)
    pos = nsel + pos_x
    gidx = base + i * 16 + iota
    plsc.store_scatter(selb_v, [pos], b, mask=sel)
    plsc.store_scatter(seli_v, [pos], gidx, mask=sel)
    return nsel + jnp.sum(sel_i), taken + jnp.sum(eq_i)
  nsel, _ = lax.fori_loop(0, VPS, p4, (_i32(0), _i32(0)))

  nch = (nsel + 15) >> 4

  def p6(k, _):
    pos = my_off + k * 16 + iota
    ok = (k * 16 + iota) < nsel
    tgt_v[...] = jnp.where(ok, pos, CAND + sid)
    pltpu.sync_copy(selb_v.at[pl.ds(k * 16, 16)], candb_sh.at[tgt_v])
    pltpu.sync_copy(seli_v.at[pl.ds(k * 16, 16)], candi_sh.at[tgt_v])
    return 0
  lax.fori_loop(0, nch, p6, 0)
  plsc.subcore_barrier()

  pltpu.sync_copy(candb_sh.at[pl.ds(0, CAND)], candb_v.at[pl.ds(0, CAND)])
  pltpu.sync_copy(candi_sh.at[pl.ds(0, CAND)], candi_v.at[pl.ds(0, CAND)])

  mybase = sid * 32
  mb0 = candb_v[pl.ds(mybase, 16)]
  mi0 = candi_v[pl.ds(mybase, 16)]
  mb1 = candb_v[pl.ds(mybase + 16, 16)]
  mi1 = candi_v[pl.ds(mybase + 16, 16)]

  def p7(p, carry):
    r0, r1 = carry
    bp = candb_v[pl.ds(p, 16)][0]
    ip = candi_v[pl.ds(p, 16)][0]
    a0 = (bp > mb0) | ((bp == mb0) & (ip < mi0))
    a1 = (bp > mb1) | ((bp == mb1) & (ip < mi1))
    return r0 + a0.astype(_i32), r1 + a1.astype(_i32)
  r0, r1 = lax.fori_loop(0, kk, p7, (zeros16, zeros16))
  v0 = (mybase + iota) < kk
  v1 = (mybase + 16 + iota) < kk
  rank_v[pl.ds(0, 16)] = jnp.where(v0, r0, 544 + sid)
  rank_v[pl.ds(16, 16)] = jnp.where(v1, r1, 544 + sid)

  plsc.store_scatter(rowbuf_v, [iota, zeros16], mi0)
  plsc.store_scatter(rowbuf_v, [16 + iota, zeros16], mi1)
  pltpu.sync_copy(rowbuf_v, sorted_sh.at[rank_v])
  plsc.subcore_barrier()

  pltpu.sync_copy(sorted_sh.at[pl.ds(mybase, 32)], obuf_v)

  for h in range(2):
    rows = iota + 16 * h
    iv = plsc.load_gather(obuf_v, [rows, zeros16])
    vmask = (mybase + 16 * h + iota) < kk
    ibuf_v[pl.ds(h * 16, 16)] = jnp.where(vmask, iv, 0)

  pltpu.async_copy(boxes_hbm.at[ibuf_v], gath_v, sem).wait()

  for h in range(2):
    rows = iota + 16 * h
    vmask = (mybase + 16 * h + iota) < kk
    sv = plsc.load_gather(gath_v, [rows, jnp.full((16,), 4, _i32)])
    sbuf_v[pl.ds(h * 16, 16)] = jnp.where(vmask, sv, _f32(NEG_INF))
    for c in range(4):
      col = plsc.load_gather(gath_v, [rows, jnp.full((16,), c, _i32)])
      colb_v[pl.ds(32 * c + 16 * h, 16)] = col

  pltpu.sync_copy(sbuf_v, osc_hbm.at[sid])
  pltpu.sync_copy(gath_v, obx_hbm.at[pl.ds(mybase, 32)])
  for c in range(4):
    pltpu.sync_copy(colb_v.at[pl.ds(32 * c, 32)],
                    obt_hbm.at[c, pl.ds(mybase, 32)])


def _make_sc_kernel():
  mesh = plsc.VectorSubcoreMesh(core_axis_name="c", subcore_axis_name="s")
  return pl.kernel(
      _sc_body,
      out_type=[
          jax.ShapeDtypeStruct((NSUB, 32), _f32),
          jax.ShapeDtypeStruct((CAND, 16), _f32),
          jax.ShapeDtypeStruct((4, CAND), _f32),
      ],
      mesh=mesh,
      compiler_params=pltpu.CompilerParams(needs_layout_passes=False,
                                           use_tc_tiling_on_sc=False),
      scratch_types=[
          pltpu.VMEM((PER_SUB,), _i32),
          pltpu.VMEM((16, 16), _i32),
          pltpu.VMEM((16,), _i32),
          pltpu.VMEM((16, 16), _i32),
          pltpu.VMEM((448,), _i32),
          pltpu.VMEM((448,), _i32),
          pltpu.VMEM((CAND + 16,), _i32),
          pltpu.VMEM((CAND + 16,), _i32),
          pltpu.VMEM((32, 16), _i32),
          pltpu.VMEM((32,), _i32),
          pltpu.VMEM((32, 16), _i32),
          pltpu.VMEM((32,), _i32),
          pltpu.VMEM((32,), _f32),
          pltpu.VMEM((32, 16), _f32),
          pltpu.VMEM((128,), _f32),
          pltpu.VMEM_SHARED((64, 16), _i32),
          pltpu.VMEM_SHARED((16, 16), _i32),
          pltpu.VMEM_SHARED((576,), _i32),
          pltpu.VMEM_SHARED((576,), _i32),
          pltpu.VMEM_SHARED((560, 16), _i32),
          pltpu.SemaphoreType.DMA,
      ],
  )


def _tc_body(sc_ref, bx_ref, bt_ref, out_ref, iou_ref):
  x1c = bx_ref[:, 0:1]
  y1c = bx_ref[:, 1:2]
  x2c = bx_ref[:, 2:3]
  y2c = bx_ref[:, 3:4]
  x1r = bt_ref[0:1, :]
  y1r = bt_ref[1:2, :]
  x2r = bt_ref[2:3, :]
  y2r = bt_ref[3:4, :]
  area_c = (x2c - x1c) * (y2c - y1c)
  area_r = (x2r - x1r) * (y2r - y1r)
  wx = jnp.clip(jnp.minimum(x2c, x2r) - jnp.maximum(x1c, x1r), 0.0, None)
  wy = jnp.clip(jnp.minimum(y2c, y2r) - jnp.maximum(y1c, y1r), 0.0, None)
  inter = wx * wy
  iou_ref[...] = inter / (area_c + area_r - inter + 1e-9)

  lane = lax.broadcasted_iota(_i32, (1, CAND), 1)

  def nms_step(i, sup):
    row = iou_ref[pl.ds(i, 1), :]
    sup_i = jnp.sum(jnp.where(lane == i, sup, 0.0))
    new = jnp.where((row > NMS_THRESH) & (lane > i) & (sup_i == 0.0),
                    1.0, 0.0)
    return jnp.maximum(sup, new)
  sup = lax.fori_loop(0, CAND, nms_step, jnp.zeros((1, CAND), _f32))

  s_row = sc_ref[...]
  keptf = jnp.where((s_row > _f32(NEG_INF)) & (sup == 0.0), 1.0, 0.0)

  r_i = lax.broadcasted_iota(_i32, (CAND, CAND), 0)
  c_i = lax.broadcasted_iota(_i32, (CAND, CAND), 1)
  le = jnp.where(r_i <= c_i, 1.0, 0.0).astype(_f32)
  pos = jnp.dot(keptf, le, preferred_element_type=_f32,
                precision=lax.Precision.HIGHEST)

  o_i = lax.broadcasted_iota(_i32, (256, CAND), 0).astype(_f32)
  oh = jnp.where((pos - 1.0 == o_i) & (keptf > 0.0), 1.0, 0.0)
  data = bx_ref[:, 0:8]
  data = jnp.where(data == _f32(NEG_INF), 0.0, data)
  out_ref[...] = jnp.dot(oh, data, preferred_element_type=_f32,
                         precision=lax.Precision.HIGHEST)


@jax.jit
def kernel(boxes, scores):
  scores_p = jnp.concatenate([scores, jnp.zeros((NPAD - N,), _f32)])
  sbits = jax.lax.bitcast_convert_type(scores_p, _i32)
  boxes16 = (jnp.zeros((NPAD, 16), _f32)
             .at[:N, :4].set(boxes)
             .at[:NPAD, 4].set(scores_p))

  osc, obx, obt = _make_sc_kernel()(sbits, boxes16)
  sc_row = osc.reshape(1, CAND)

  out = pl.pallas_call(
      _tc_body,
      out_shape=jax.ShapeDtypeStruct((256, 8), _f32),
      scratch_shapes=[pltpu.VMEM((CAND, CAND), _f32)],
  )(sc_row, obx, obt)
  return out[:DET, :5]
